# speedup vs baseline: 1.4844x; 1.4844x over previous
import jax
import jax.numpy as jnp
from jax.experimental import pallas as pl

_S_BLK = 512


def _add_body(x_ref, t_ref, o_ref):
    o_ref[...] = x_ref[...] + t_ref[...]


def kernel(x, table):
    B, S, D = x.shape
    grid = (S // _S_BLK, B)
    return pl.pallas_call(
        _add_body,
        grid=grid,
        in_specs=[
            pl.BlockSpec((1, _S_BLK, D), lambda s, b: (b, s, 0)),
            pl.BlockSpec((_S_BLK, D), lambda s, b: (s, 0)),
        ],
        out_specs=pl.BlockSpec((1, _S_BLK, D), lambda s, b: (b, s, 0)),
        out_shape=jax.ShapeDtypeStruct(x.shape, x.dtype),
    )(x, table)
